# Initial kernel scaffold; baseline (speedup 1.0000x reference)
import jax
import jax.numpy as jnp
from jax.experimental import pallas as pl


def kernel(x, edge_index, batch, W1, b1, W2, b2, W3, b3, Wfc, bfc):
    raise NotImplementedError("write your pallas kernel here")


# speedup vs baseline: 8.3600x; 8.3600x over previous
import functools

import jax
import jax.numpy as jnp
from jax import lax
from jax.experimental import pallas as pl
from jax.experimental.pallas import tpu as pltpu
from jax.experimental.pallas import tpu_sc as plsc

NN = 10000
NPAD = 10240
EE = 160000
G = 64
K = 10
L = 16
NSUB = 16
NCORE = 2
RB = 256
F32 = jnp.float32
I32 = jnp.int32

EPT = EE // NSUB
PB = 128
NBATCH = (EPT + PB - 1) // PB
EPT_PAD = NBATCH * PB
DEPT = EE // (NSUB * NCORE)
DNB = (DEPT + PB - 1) // PB
DEPT_PAD = DNB * PB
RPT = NPAD // NSUB

_mesh = lambda: plsc.VectorSubcoreMesh(core_axis_name="c", subcore_axis_name="s")
_SC_PARAMS = pltpu.CompilerParams(needs_layout_passes=False)


def _fill_zeros(ref, n16):
    z = jnp.zeros((L,), F32)
    for j in range(n16):
        ref[pl.ds(j * L, L)] = z


@functools.partial(
    pl.kernel,
    out_type=jax.ShapeDtypeStruct((NCORE, NPAD), F32),
    mesh=_mesh(),
    scratch_types=[
        pltpu.VMEM((DNB, PB), I32),
        pltpu.VMEM((PB,), F32),
        pltpu.VMEM((RPT,), F32),
        pltpu.VMEM_SHARED((NPAD,), F32),
    ],
    compiler_params=_SC_PARAMS,
)
def _deg_kernel(dst_hbm, out_hbm, idx_v, ones_v, zero_v, acc):
    c = lax.axis_index("c")
    s = lax.axis_index("s")
    w = s * NCORE + c
    pltpu.sync_copy(dst_hbm.at[w], idx_v)
    one = jnp.ones((L,), F32)
    for j in range(PB // L):
        ones_v[pl.ds(j * L, L)] = one
    _fill_zeros(zero_v, RPT // L)
    pltpu.sync_copy(zero_v, acc.at[pl.ds(s * RPT, RPT)])
    plsc.subcore_barrier()

    def body(j, carry):
        pltpu.sync_copy(ones_v, acc.at[idx_v.at[j]], add=True)
        return carry

    lax.fori_loop(0, DNB, body, 0)
    plsc.subcore_barrier()

    @pl.when(s == 0)
    def _():
        pltpu.sync_copy(acc, out_hbm.at[c])


def _make_prop(nb):
    nphase = nb // NCORE
    out_t = tuple(jax.ShapeDtypeStruct((NPAD, 128), F32) for _ in range(nb))
    scratch = [
        pltpu.VMEM((NBATCH, PB), I32),
        pltpu.VMEM((NBATCH, PB), I32),
        pltpu.VMEM((PB, 128), F32),
        pltpu.VMEM((L, 128), F32),
        pltpu.VMEM_SHARED((NPAD, 128), F32),
        pltpu.SemaphoreType.DMA,
    ]

    @functools.partial(pl.kernel, out_type=out_t, mesh=_mesh(),
                       scratch_types=scratch, compiler_params=_SC_PARAMS)
    def prop(*refs):
        q = refs[:nb]
        src_hbm, dst_hbm = refs[nb], refs[nb + 1]
        outs = refs[nb + 2: 2 * nb + 2]
        src_v, dst_v, rows_v, ztile, acc, sem = refs[2 * nb + 2:]
        c = lax.axis_index("c")
        s = lax.axis_index("s")
        pltpu.sync_copy(src_hbm.at[s], src_v)
        pltpu.sync_copy(dst_hbm.at[s], dst_v)
        z = jnp.zeros((L,), F32)
        for r in range(L):
            for j in range(128 // L):
                ztile[r, pl.ds(j * L, L)] = z

        for phase in range(nphase):
            def zbody(j, carry):
                pltpu.sync_copy(ztile, acc.at[pl.ds(s * RPT + j * L, L)])
                return carry
            lax.fori_loop(0, RPT // L, zbody, 0)
            plsc.subcore_barrier()
            for cc in range(NCORE):
                blk = cc * nphase + phase

                @pl.when(c == cc)
                def _(blk=blk):
                    def body(j, carry):
                        pltpu.async_copy(q[blk].at[src_v.at[j]], rows_v,
                                         sem).wait()
                        pltpu.sync_copy(rows_v, acc.at[dst_v.at[j]], add=True)
                        return carry
                    lax.fori_loop(0, NBATCH, body, 0)
            plsc.subcore_barrier()
            for cc in range(NCORE):
                blk = cc * nphase + phase

                @pl.when(c == cc)
                def _(blk=blk):
                    pltpu.sync_copy(acc.at[pl.ds(s * RPT, RPT)],
                                    outs[blk].at[pl.ds(s * RPT, RPT)])
            plsc.subcore_barrier()

    return prop


_prop2 = _make_prop(2)
_prop4 = _make_prop(4)


@functools.partial(
    pl.kernel,
    out_type=tuple(jax.ShapeDtypeStruct((G * L, 128), F32) for _ in range(4)),
    mesh=_mesh(),
    scratch_types=[
        pltpu.VMEM((NPAD,), F32),
        pltpu.VMEM((G,), I32),
        pltpu.VMEM((G,), I32),
        pltpu.VMEM((L,), I32),
        pltpu.VMEM((L, 128), F32),
        pltpu.SemaphoreType.DMA,
    ],
    compiler_params=_SC_PARAMS,
)
def _topk_kernel(score_hbm, starts_hbm, ends_hbm, c0, c1, c2, c3,
                 a0, a1, a2, a3, sbuf, st_v, en_v, idx_v, rows_v, sem):
    cblks = (c0, c1, c2, c3)
    aggs = (a0, a1, a2, a3)
    c = lax.axis_index("c")
    s = lax.axis_index("s")
    wid = s * NCORE + c
    pltpu.sync_copy(score_hbm, sbuf)
    pltpu.sync_copy(starts_hbm, st_v)
    pltpu.sync_copy(ends_hbm, en_v)
    iota = lax.iota(I32, L)
    NEG = jnp.float32(-3.0e38)
    BIG = jnp.int32(1 << 30)

    for gl in range(2):
        g = wid * 2 + gl
        base = (g // L) * L
        lane = g - base
        selm = iota == lane
        start = jnp.sum(jnp.where(selm, st_v[pl.ds(base, L)], 0))
        end = jnp.sum(jnp.where(selm, en_v[pl.ds(base, L)], 0))
        kk = jnp.minimum(end - start, K)
        ch0 = start // L
        ch1 = (end + L - 1) // L

        def slot_body(slot, tk):
            def mx(cb, m):
                v = sbuf[pl.ds(cb * L, L)]
                gi = cb * L + iota
                vm = jnp.where((gi >= start) & (gi < end), v, NEG)
                return jnp.maximum(m, jnp.max(vm))
            M = lax.fori_loop(ch0, ch1, mx, NEG)

            def fx(cb, f):
                v = sbuf[pl.ds(cb * L, L)]
                gi = cb * L + iota
                vm = jnp.where((gi >= start) & (gi < end), v, NEG)
                ff = plsc.all_reduce_ffs(vm >= M)
                ff = jnp.max(ff) if ff.ndim else ff
                cand = jnp.where(ff < L, cb * L + ff, BIG)
                return jnp.minimum(f, cand)
            found = lax.fori_loop(ch0, ch1, fx, BIG)
            cb = found // L
            ln = found - cb * L
            row = sbuf[pl.ds(cb * L, L)]
            sbuf[pl.ds(cb * L, L)] = jnp.where(iota == ln, NEG, row)
            return jnp.where(iota == slot, found, tk)

        tk = lax.fori_loop(0, kk, slot_body, jnp.full((L,), NN, I32))
        idx_v[...] = tk
        for fb in range(4):
            pltpu.async_copy(cblks[fb].at[idx_v], rows_v, sem).wait()
            pltpu.sync_copy(rows_v, aggs[fb].at[pl.ds(g * L, L)])


def _pre_body(x_r, da_r, db_r, dinv_r, q0_r, q1_r):
    i = pl.program_id(0)
    deg = da_r[...] + db_r[...] + 1.0
    dinv = lax.rsqrt(deg)
    valid = lax.broadcasted_iota(I32, (RB, 1), 0) + i * RB < NN
    dinv_r[...] = jnp.where(valid, dinv, 1.0)
    q = jnp.where(valid, x_r[...] * dinv, 0.0)
    q0_r[...] = q[:, :128]
    q1_r[...] = q[:, 128:]


def _pre_tc(x, degA, degB):
    grid = NPAD // RB
    bs_col = pl.BlockSpec((RB, 1), lambda i: (i, 0))
    bs_blk = pl.BlockSpec((RB, 128), lambda i: (i, 0))
    return pl.pallas_call(
        _pre_body,
        grid=(grid,),
        in_specs=[pl.BlockSpec((RB, 256), lambda i: (i, 0)), bs_col, bs_col],
        out_specs=[bs_col, bs_blk, bs_blk],
        out_shape=[jax.ShapeDtypeStruct((NPAD, 1), F32),
                   jax.ShapeDtypeStruct((NPAD, 128), F32),
                   jax.ShapeDtypeStruct((NPAD, 128), F32)],
    )(x, degA, degB)


def _layer1_body(s0, s1, q0, q1, dinv_r, W1_r, b1_r, W2_r, *outs):
    i = pl.program_id(0)
    dinv = dinv_r[...]
    p = jnp.concatenate(
        [dinv * (s0[...] + q0[...]), dinv * (s1[...] + q1[...])], axis=1)
    h = jax.nn.relu(jnp.dot(p, W1_r[...], preferred_element_type=F32)
                    + b1_r[...])
    q = dinv * jnp.dot(h, W2_r[...], preferred_element_type=F32)
    valid = lax.broadcasted_iota(I32, (RB, 1), 0) + i * RB < NN
    q = jnp.where(valid, q, 0.0)
    for fb in range(4):
        outs[fb][...] = q[:, fb * 128:(fb + 1) * 128]


def _layer1_tc(s0, s1, q0, q1, dinv, W1, b1, W2):
    grid = NPAD // RB
    bs_blk = pl.BlockSpec((RB, 128), lambda i: (i, 0))
    bs_col = pl.BlockSpec((RB, 1), lambda i: (i, 0))
    full = lambda a, b: pl.BlockSpec((a, b), lambda i: (0, 0))
    return pl.pallas_call(
        _layer1_body,
        grid=(grid,),
        in_specs=[bs_blk, bs_blk, bs_blk, bs_blk, bs_col,
                  full(256, 512), full(1, 512), full(512, 512)],
        out_specs=[bs_blk] * 4,
        out_shape=[jax.ShapeDtypeStruct((NPAD, 128), F32)] * 4,
    )(s0, s1, q0, q1, dinv, W1, b1, W2)


def _layer2_body(s0, s1, s2, s3, q0, q1, q2, q3, dinv_r, b_r, W_r, *outs):
    i = pl.program_id(0)
    dinv = dinv_r[...]
    ss = (s0, s1, s2, s3)
    qq = (q0, q1, q2, q3)
    pre = jnp.concatenate(
        [dinv * (ss[fb][...] + qq[fb][...]) for fb in range(4)], axis=1)
    h = jax.nn.relu(pre + b_r[...])
    q = dinv * jnp.dot(h, W_r[...], preferred_element_type=F32)
    valid = lax.broadcasted_iota(I32, (RB, 1), 0) + i * RB < NN
    q = jnp.where(valid, q, 0.0)
    for fb in range(4):
        outs[fb][...] = q[:, fb * 128:(fb + 1) * 128]


def _layer2_tc(s_blks, q_blks, dinv, b2, W3):
    grid = NPAD // RB
    bs_blk = pl.BlockSpec((RB, 128), lambda i: (i, 0))
    bs_col = pl.BlockSpec((RB, 1), lambda i: (i, 0))
    full = lambda a, b: pl.BlockSpec((a, b), lambda i: (0, 0))
    return pl.pallas_call(
        _layer2_body,
        grid=(grid,),
        in_specs=[bs_blk] * 8 + [bs_col, full(1, 512), full(512, 512)],
        out_specs=[bs_blk] * 4,
        out_shape=[jax.ShapeDtypeStruct((NPAD, 128), F32)] * 4,
    )(*s_blks, *q_blks, dinv, b2, W3)


def _layer3_body(s0, s1, s2, s3, q0, q1, q2, q3, dinv_r, b_r, batch_r, U_r,
                 c0, c1, c2, c3, score_r, counts_r, starts_r, ends_r):
    i = pl.program_id(0)
    ni = pl.num_programs(0)
    dinv = dinv_r[...]
    ss = (s0, s1, s2, s3)
    qq = (q0, q1, q2, q3)
    valid = lax.broadcasted_iota(I32, (RB, 1), 0) + i * RB < NN
    outs = (c0, c1, c2, c3)
    for fb in range(4):
        cb = jax.nn.relu(dinv * (ss[fb][...] + qq[fb][...])
                         + b_r[...][:, fb * 128:(fb + 1) * 128])
        cb = jnp.where(valid, cb, 0.0)
        outs[fb][...] = cb
        if fb == 3:
            score_r[...] = cb[:, 127:128]
    oh = (batch_r[...] == lax.broadcasted_iota(I32, (1, G), 1)).astype(F32)
    cnt = jnp.sum(oh, axis=0, keepdims=True)

    @pl.when(i == 0)
    def _():
        counts_r[...] = cnt

    @pl.when(i > 0)
    def _():
        counts_r[...] += cnt

    @pl.when(i == ni - 1)
    def _():
        tot = counts_r[...]
        st = jnp.dot(tot, U_r[...], preferred_element_type=F32)
        starts_r[...] = st.astype(I32)
        ends_r[...] = (st + tot).astype(I32)


def _layer3_tc(s_blks, q_blks, dinv, b3, batch_col, U):
    grid = NPAD // RB
    bs_blk = pl.BlockSpec((RB, 128), lambda i: (i, 0))
    bs_col = pl.BlockSpec((RB, 1), lambda i: (i, 0))
    full = lambda a, b: pl.BlockSpec((a, b), lambda i: (0, 0))
    return pl.pallas_call(
        _layer3_body,
        grid=(grid,),
        in_specs=[bs_blk] * 8 + [bs_col, full(1, 512), bs_col, full(G, G)],
        out_specs=[bs_blk] * 4 + [bs_col, full(1, G), full(1, G), full(1, G)],
        out_shape=[jax.ShapeDtypeStruct((NPAD, 128), F32)] * 4
        + [jax.ShapeDtypeStruct((NPAD, 1), F32),
           jax.ShapeDtypeStruct((1, G), F32),
           jax.ShapeDtypeStruct((1, G), I32),
           jax.ShapeDtypeStruct((1, G), I32)],
    )(*s_blks, *q_blks, dinv, b3, batch_col, U)


def _final_body(a0, a1, a2, a3, w0, w1, w2, w3, bfc_r, batch_r,
                mix_r, mixp_r, probg_r, logits_s, probs_s):
    i = pl.program_id(0)
    aa = (a0, a1, a2, a3)
    ww = (w0, w1, w2, w3)

    @pl.when(i == 0)
    def _():
        lg = bfc_r[...]
        for fb in range(4):
            lg = lg + jnp.dot(aa[fb][...], ww[fb][...],
                              preferred_element_type=F32)
        logits_s[...] = lg
        m = jnp.max(lg, axis=1, keepdims=True)
        e = jnp.exp(lg - m)
        pr = e / jnp.sum(e, axis=1, keepdims=True)
        probs_s[...] = pr
        probg_r[...] = pr

    oh = (batch_r[...] == lax.broadcasted_iota(I32, (1, G), 1)).astype(F32)
    mix_r[...] = jnp.dot(oh, logits_s[...], preferred_element_type=F32)
    mixp_r[...] = jnp.dot(oh, probs_s[...], preferred_element_type=F32)


def _final_tc(agg_r, wfc_r, bfc, batch_col):
    grid = NPAD // RB
    bs_col = pl.BlockSpec((RB, 1), lambda i: (i, 0))
    bs_out = pl.BlockSpec((RB, 16), lambda i: (i, 0))
    full = lambda a, b: pl.BlockSpec((a, b), lambda i: (0, 0))
    return pl.pallas_call(
        _final_body,
        grid=(grid,),
        in_specs=[full(G, L * 128)] * 4 + [full(L * 128, 16)] * 4
        + [full(1, 16), bs_col],
        out_specs=[bs_out, bs_out, full(G, 16)],
        out_shape=[jax.ShapeDtypeStruct((NN, 16), F32),
                   jax.ShapeDtypeStruct((NN, 16), F32),
                   jax.ShapeDtypeStruct((G, 16), F32)],
        scratch_shapes=[pltpu.VMEM((G, 16), F32), pltpu.VMEM((G, 16), F32)],
    )(*agg_r, *wfc_r, bfc, batch_col)


def kernel(x, edge_index, batch, W1, b1, W2, b2, W3, b3, Wfc, bfc):
    src = edge_index[0]
    dst = edge_index[1]
    src_p = jnp.pad(src.reshape(NSUB, EPT), ((0, 0), (0, EPT_PAD - EPT)),
                    constant_values=NN).reshape(NSUB, NBATCH, PB)
    dst_p = jnp.pad(dst.reshape(NSUB, EPT), ((0, 0), (0, EPT_PAD - EPT)),
                    constant_values=NN).reshape(NSUB, NBATCH, PB)
    dst_d = jnp.pad(dst.reshape(NSUB * NCORE, DEPT),
                    ((0, 0), (0, DEPT_PAD - DEPT)),
                    constant_values=NN).reshape(NSUB * NCORE, DNB, PB)
    batch_col = jnp.pad(batch, (0, NPAD - NN),
                        constant_values=G).reshape(NPAD, 1)
    b1r = b1.reshape(1, 512)
    b2r = b2.reshape(1, 512)
    b3r = b3.reshape(1, 512)
    bfcr = bfc.reshape(1, 16)
    U = jnp.triu(jnp.ones((G, G), F32), 1)
    wfc_r = Wfc.reshape(K, 4, 128, 16).transpose(1, 0, 2, 3)
    wfc_r = [jnp.pad(wfc_r[fb], ((0, L - K), (0, 0), (0, 0))
                     ).reshape(L * 128, 16) for fb in range(4)]

    deg = _deg_kernel(dst_d)
    degA = deg[0].reshape(NPAD, 1)
    degB = deg[1].reshape(NPAD, 1)
    dinv, q00, q01 = _pre_tc(x, degA, degB)

    s00, s01 = _prop2(q00, q01, src_p, dst_p)
    q1 = _layer1_tc(s00, s01, q00, q01, dinv, W1, b1r, W2)
    s1 = _prop4(*q1, src_p, dst_p)
    q2 = _layer2_tc(s1, q1, dinv, b2r, W3)
    s2 = _prop4(*q2, src_p, dst_p)
    c_blks, score, _counts, starts, ends = (
        lambda o: (o[:4], o[4], o[5], o[6], o[7]))(
            _layer3_tc(s2, q2, dinv, b3r, batch_col, U))

    aggs = _topk_kernel(score.reshape(NPAD), starts.reshape(G),
                        ends.reshape(G), *c_blks)
    agg_r = [a.reshape(G, L * 128) for a in aggs]
    mix, mixp, probg = _final_tc(agg_r, wfc_r, bfcr, batch_col)
    return (mix, mixp, probg)
